# baseline (speedup 1.0000x reference)
import functools
import math

import jax
import jax.numpy as jnp
from jax import lax
from jax.experimental import pallas as pl
from jax.experimental.pallas import tpu as pltpu
from jax.experimental.pallas import tpu_sc as plsc

N = 10000
E = 160000
C = 256
H = 8
HD = 32
CP = C // 2

_NC = 2
_NS = 16
_NW = _NC * _NS

_EB = 80
_EWP = 5120
_EP = _NW * _EWP
_NCH = _EWP // _EB
_NCA = 48
_NCB = 2 * _NCH - _NCA
_WCH = _EB * H
_RC = 2 * _EB
_RSC = 10240
_RPT = _RSC // _NS

_ROWBLK = 2000

_mesh = plsc.VectorSubcoreMesh(core_axis_name="c", subcore_axis_name="s")
_scp = pltpu.CompilerParams(use_tc_tiling_on_sc=False,
                            needs_layout_passes=False)


def _qkv_body(x_ref, g_ref, b_ref, wq_ref, bq_ref, wk_ref, bk_ref,
              wv_ref, bv_ref, q_ref, k_ref, v_ref):
    x = x_ref[...]
    mu = jnp.mean(x, axis=1, keepdims=True)
    var = jnp.mean((x - mu) * (x - mu), axis=1, keepdims=True)
    h = (x - mu) * lax.rsqrt(var + 1e-5) * g_ref[...] + b_ref[...]
    dn = (((1,), (1,)), ((), ()))
    q = lax.dot_general(h, wq_ref[...], dn, preferred_element_type=jnp.float32)
    k = lax.dot_general(h, wk_ref[...], dn, preferred_element_type=jnp.float32)
    v = lax.dot_general(h, wv_ref[...], dn, preferred_element_type=jnp.float32)
    q_ref[...] = (q + bq_ref[...]) * (1.0 / math.sqrt(HD))
    k_ref[...] = k + bk_ref[...]
    v_ref[...] = (v + bv_ref[...]).astype(jnp.bfloat16)


def _qkv(x, ln1_g, ln1_b, Wq, bq, Wk, bk, Wv, bv):
    grid = (N // _ROWBLK,)
    full = pl.BlockSpec((C, C), lambda i: (0, 0))
    vec = pl.BlockSpec((1, C), lambda i: (0, 0))
    blk = pl.BlockSpec((_ROWBLK, C), lambda i: (i, 0))
    return pl.pallas_call(
        _qkv_body,
        grid=grid,
        in_specs=[blk, vec, vec, full, vec, full, vec, full, vec],
        out_specs=[blk, blk, blk],
        out_shape=[jax.ShapeDtypeStruct((N, C), jnp.float32),
                   jax.ShapeDtypeStruct((N, C), jnp.float32),
                   jax.ShapeDtypeStruct((N, C), jnp.bfloat16)],
    )(x, ln1_g, ln1_b, Wq, bq, Wk, bk, Wv, bv)


@functools.partial(
    pl.kernel,
    out_type=(
        jax.ShapeDtypeStruct((H * _EP,), jnp.float32),
        jax.ShapeDtypeStruct((_NW, H * 16), jnp.float32),
    ),
    mesh=_mesh,
    scratch_types=[
        [pltpu.VMEM((_RC,), jnp.int32) for _ in range(2)],
        [pltpu.VMEM((_EB, CP), jnp.float32) for _ in range(4)],
        [pltpu.VMEM((_WCH,), jnp.float32) for _ in range(2)],
        pltpu.VMEM((H * 16,), jnp.float32),
        [pltpu.SemaphoreType.DMA for _ in range(2)],
    ],
    compiler_params=_scp,
)
def _edge_w(rc_hbm, q_hbm, k_hbm, w_hbm, dpart_hbm,
            rcs, qks, wbufs, dacc, gsems):
    cid = lax.axis_index("c")
    sid = lax.axis_index("s")
    wid = cid * _NS + sid
    zero16 = jnp.zeros((16,), jnp.float32)
    for h in range(H):
        dacc[pl.ds(h * 16, 16)] = zero16
    lane = lax.iota(jnp.int32, 16)
    crot = [(lane + t) & 15 for t in range(16)]
    nch = jnp.where(cid == 0, _NCA, _NCB)
    cg0 = jnp.where(cid == 0, sid * _NCA, _NS * _NCA + sid * _NCB)

    def issue(c, vs):
        pltpu.sync_copy(rc_hbm.at[pl.ds((cg0 + c) * _RC, _RC)], rcs[vs])
        pltpu.async_copy(q_hbm.at[rcs[vs].at[pl.ds(0, _EB)]],
                         qks[2 * vs], gsems[vs])
        pltpu.async_copy(k_hbm.at[rcs[vs].at[pl.ds(_EB, _EB)]],
                         qks[2 * vs + 1], gsems[vs])

    def gath_wait(vs):
        pltpu.make_async_copy(q_hbm.at[rcs[vs].at[pl.ds(0, _EB)]],
                              qks[2 * vs], gsems[vs]).wait()
        pltpu.make_async_copy(k_hbm.at[rcs[vs].at[pl.ds(_EB, _EB)]],
                              qks[2 * vs + 1], gsems[vs]).wait()

    def compute(c, vs):
        qrows = qks[2 * vs]
        krows = qks[2 * vs + 1]
        wbuf = wbufs[vs]
        ebase = (cg0 + c) * _EB

        def grp(g, c2):
            eloc = g * 16 + lane
            mask = (ebase + eloc) < E
            for h in range(H):
                a0 = zero16
                a1 = zero16
                a2 = zero16
                a3 = zero16
                for d in range(CP // H):
                    cvec = crot[d] + (h * (CP // H))
                    qg = plsc.load_gather(qrows, [eloc, cvec])
                    kg = plsc.load_gather(krows, [eloc, cvec])
                    prod = (plsc.bitcast(qg, jnp.bfloat16)
                            * plsc.bitcast(kg, jnp.bfloat16))
                    pa, pb = plsc.unpack(prod,
                                         format=plsc.PackFormat.INTERLEAVED)
                    if d % 2 == 0:
                        a0 = a0 + pa
                        a1 = a1 + pb
                    else:
                        a2 = a2 + pa
                        a3 = a3 + pb
                acc = (a0 + a1) + (a2 + a3)
                wv = jnp.where(mask, jnp.exp(acc), 0.0)
                sl = pl.ds(h * 16, 16)
                dacc[sl] = dacc[sl] + wv
                wbuf[pl.ds(h * _EB + g * 16, 16)] = wv
            return c2

        lax.fori_loop(0, _EB // 16, grp, 0)
        pltpu.sync_copy(wbuf, w_hbm.at[pl.ds((cg0 + c) * _WCH, _WCH)])

    issue(0, 0)
    issue(1, 1)

    def body(j, carry):
        gath_wait(0)
        compute(2 * j, 0)
        issue((2 * j + 2) % nch, 0)
        gath_wait(1)
        compute(2 * j + 1, 1)
        issue((2 * j + 3) % nch, 1)
        return carry

    lax.fori_loop(0, nch // 2, body, 0)
    gath_wait(0)
    gath_wait(1)
    pltpu.sync_copy(dacc, dpart_hbm.at[wid])


@functools.partial(
    pl.kernel,
    out_type=jax.ShapeDtypeStruct((_NC * _RSC, C), jnp.bfloat16),
    mesh=_mesh,
    scratch_types=[
        [pltpu.VMEM((_RC,), jnp.int32) for _ in range(2)],
        [pltpu.VMEM((_WCH,), jnp.float32) for _ in range(2)],
        [pltpu.VMEM((_EB,), jnp.int32) for _ in range(2)],
        [pltpu.VMEM((_EB, C), jnp.bfloat16) for _ in range(2)],
        [pltpu.VMEM((_EB, C), jnp.bfloat16) for _ in range(2)],
        pltpu.VMEM((16, C), jnp.bfloat16),
        pltpu.VMEM_SHARED((_RSC, C), jnp.bfloat16),
        [pltpu.SemaphoreType.DMA for _ in range(2)],
        [pltpu.SemaphoreType.DMA for _ in range(2)],
    ],
    compiler_params=_scp,
)
def _edge_scatter(rc_hbm, v_hbm, w_hbm, out_hbm,
                  rcs, wchs, rowvs, vrows, wvs, zbuf, acc,
                  gsems, ssems):
    cid = lax.axis_index("c")
    sid = lax.axis_index("s")
    lane = lax.iota(jnp.int32, 16)
    zero32 = jnp.zeros((32,), jnp.bfloat16)
    nch = jnp.where(cid == 0, _NCA, _NCB)
    cg0 = jnp.where(cid == 0, sid * _NCA, _NS * _NCA + sid * _NCB)

    def issue(c, vs):
        pltpu.sync_copy(rc_hbm.at[pl.ds((cg0 + c) * _RC, _RC)], rcs[vs])
        pltpu.sync_copy(w_hbm.at[pl.ds((cg0 + c) * _WCH, _WCH)], wchs[vs])
        pltpu.async_copy(v_hbm.at[rcs[vs].at[pl.ds(_EB, _EB)]],
                         vrows[vs], gsems[vs])

    def gath_wait(vs):
        pltpu.make_async_copy(v_hbm.at[rcs[vs].at[pl.ds(_EB, _EB)]],
                              vrows[vs], gsems[vs]).wait()

    def scat_wait(vs):
        pltpu.make_async_copy(wvs[vs], acc.at[rowvs[vs]], ssems[vs]).wait()

    def process(vs):
        for t in range(_EB // 16):
            rowvs[vs][pl.ds(t * 16, 16)] = rcs[vs][pl.ds(t * 16, 16)]
        wch = wchs[vs]
        wvbuf = wvs[vs]
        vr = vrows[vs]

        def grp(g, c2):
            for h in range(H):
                w16 = wch[pl.ds(h * _EB + g * 16, 16)]
                for e16 in range(16):
                    e = g * 16 + e16
                    wb = jnp.broadcast_to(w16[e16], (16,))
                    wbb = plsc.pack(wb, wb, format=plsc.PackFormat.INTERLEAVED)
                    sl = pl.ds(h * HD, 32)
                    wvbuf[e, sl] = vr[e, sl] * wbb
            return c2

        lax.fori_loop(0, _EB // 16, grp, 0)
        pltpu.async_copy(wvbuf, acc.at[rowvs[vs]], ssems[vs], add=True)

    for r in range(16):
        for j2 in range(C // 32):
            zbuf[r, pl.ds(j2 * 32, 32)] = zero32

    issue(0, 0)
    issue(1, 1)

    def zinit(t, carry):
        pltpu.sync_copy(zbuf, acc.at[pl.ds(sid * _RPT + t * 16, 16)])
        return carry

    lax.fori_loop(0, _RPT // 16, zinit, 0)
    plsc.subcore_barrier()

    def body(j, carry):
        @pl.when(j > 0)
        def _():
            scat_wait(0)
            scat_wait(1)

        gath_wait(0)
        process(0)
        issue((2 * j + 2) % nch, 0)
        gath_wait(1)
        process(1)
        issue((2 * j + 3) % nch, 1)
        return carry

    lax.fori_loop(0, nch // 2, body, 0)
    gath_wait(0)
    gath_wait(1)
    scat_wait(0)
    scat_wait(1)
    plsc.subcore_barrier()

    def wout(t, carry):
        sl = pl.ds(sid * _RPT + t * 16, 16)
        pltpu.sync_copy(acc.at[sl],
                        out_hbm.at[pl.ds(cid * _RSC + sid * _RPT + t * 16, 16)])
        return carry

    lax.fori_loop(0, _RPT // 16, wout, 0)


def _erf(x):
    a1, a2, a3 = 0.254829592, -0.284496736, 1.421413741
    a4, a5, p = -1.453152027, 1.061405429, 0.3275911
    s = jnp.sign(x)
    z = jnp.abs(x)
    t = 1.0 / (1.0 + p * z)
    y = 1.0 - (((((a5 * t + a4) * t) + a3) * t + a2) * t + a1) * t * jnp.exp(-z * z)
    return s * y


def _gelu(x):
    return 0.5 * x * (1.0 + _erf(x * (1.0 / math.sqrt(2.0))))


def _epi_body(a0_ref, a1_ref, x_ref, dp_ref, grp_ref, sel_ref, wo_ref, bo_ref,
              g2_ref, b2_ref, w1_ref, b1_ref, w2_ref, bias2_ref, out_ref):
    dn = (((1,), (1,)), ((), ()))
    dnr = (((1,), (0,)), ((), ()))
    dsum = jnp.sum(dp_ref[...], axis=0, keepdims=True)
    den8 = lax.dot_general(dsum, grp_ref[...], dnr,
                           preferred_element_type=jnp.float32)
    svec = lax.dot_general(1.0 / den8, sel_ref[...], dnr,
                           preferred_element_type=jnp.float32)
    att = (a0_ref[...].astype(jnp.float32)
           + a1_ref[...].astype(jnp.float32)) * svec
    o = lax.dot_general(att, wo_ref[...], dn,
                        preferred_element_type=jnp.float32) + bo_ref[...]
    o = o + x_ref[...]
    mu = jnp.mean(o, axis=1, keepdims=True)
    var = jnp.mean((o - mu) * (o - mu), axis=1, keepdims=True)
    t = (o - mu) * lax.rsqrt(var + 1e-5) * g2_ref[...] + b2_ref[...]
    u = lax.dot_general(t, w1_ref[...], dn,
                        preferred_element_type=jnp.float32) + b1_ref[...]
    u = _gelu(u)
    y = lax.dot_general(u, w2_ref[...], dn,
                        preferred_element_type=jnp.float32) + bias2_ref[...]
    out_ref[...] = y + o


def _epilogue(a0, a1, x, dparts, grp, sel, Wo, bo, ln2_g, ln2_b, W1, b1, W2, b2):
    grid = (N // _ROWBLK,)
    blk = pl.BlockSpec((_ROWBLK, C), lambda i: (i, 0))
    vec = pl.BlockSpec((1, C), lambda i: (0, 0))
    return pl.pallas_call(
        _epi_body,
        grid=grid,
        in_specs=[
            blk, blk, blk,
            pl.BlockSpec((_NW, H * 16), lambda i: (0, 0)),
            pl.BlockSpec((H * 16, H), lambda i: (0, 0)),
            pl.BlockSpec((H, C), lambda i: (0, 0)),
            pl.BlockSpec((C, C), lambda i: (0, 0)),
            vec, vec, vec,
            pl.BlockSpec((4 * C, C), lambda i: (0, 0)),
            pl.BlockSpec((1, 4 * C), lambda i: (0, 0)),
            pl.BlockSpec((C, 4 * C), lambda i: (0, 0)),
            vec,
        ],
        out_specs=blk,
        out_shape=jax.ShapeDtypeStruct((N, C), jnp.float32),
    )(a0, a1, x, dparts, grp, sel, Wo, bo, ln2_g, ln2_b, W1, b1, W2, b2)


def _selectors():
    sel = jnp.repeat(jnp.eye(H, dtype=jnp.float32), HD, axis=1)
    grp = jnp.repeat(jnp.eye(H, dtype=jnp.float32), 16, axis=1).T
    return grp, sel


def kernel(x, edge_index, Wq, bq, Wk, bk, Wv, bv, Wo, bo,
           ln1_g, ln1_b, ln2_g, ln2_b, W1, b1, W2, b2):
    pad = jnp.zeros((_EP - E,), jnp.int32)
    row = jnp.concatenate([edge_index[0], pad])
    col = jnp.concatenate([edge_index[1], pad])
    rc = jnp.concatenate([row.reshape(-1, _EB), col.reshape(-1, _EB)],
                         axis=1).reshape(-1)
    q, k, v = _qkv(x, ln1_g.reshape(1, C), ln1_b.reshape(1, C),
                   Wq, bq.reshape(1, C), Wk, bk.reshape(1, C),
                   Wv, bv.reshape(1, C))
    qp = lax.bitcast_convert_type(
        q.astype(jnp.bfloat16).reshape(N, CP, 2), jnp.float32)
    kp = lax.bitcast_convert_type(
        k.astype(jnp.bfloat16).reshape(N, CP, 2), jnp.float32)
    w, dparts = _edge_w(rc, qp, kp)
    out_pad = _edge_scatter(rc, v, w)
    a0 = out_pad[:N]
    a1 = out_pad[_RSC:_RSC + N]
    grp, sel = _selectors()
    return _epilogue(a0, a1, x, dparts, grp, sel,
                     Wo, bo.reshape(1, C), ln2_g.reshape(1, C),
                     ln2_b.reshape(1, C), W1, b1.reshape(1, 4 * C),
                     W2, b2.reshape(1, C))


# speedup vs baseline: 1.1211x; 1.1211x over previous
import functools
import math

import jax
import jax.numpy as jnp
from jax import lax
from jax.experimental import pallas as pl
from jax.experimental.pallas import tpu as pltpu
from jax.experimental.pallas import tpu_sc as plsc

N = 10000
E = 160000
C = 256
H = 8
HD = 32
CP = C // 2

_NC = 2
_NS = 16
_NW = _NC * _NS

_EB = 80
_EWP = 5120
_EP = _NW * _EWP
_NCH = _EWP // _EB
_NCA = 80
_NCB = 2 * _NCH - _NCA
_WCH = _EB * H
_RC = 2 * _EB
_RSC = 10240
_RPT = _RSC // _NS

_ROWBLK = 2000

_mesh = plsc.VectorSubcoreMesh(core_axis_name="c", subcore_axis_name="s")
_scp = pltpu.CompilerParams(use_tc_tiling_on_sc=False,
                            needs_layout_passes=False)


def _qkv_body(x_ref, g_ref, b_ref, wq_ref, bq_ref, wk_ref, bk_ref,
              wv_ref, bv_ref, q_ref, k_ref, v_ref):
    x = x_ref[...]
    mu = jnp.mean(x, axis=1, keepdims=True)
    var = jnp.mean((x - mu) * (x - mu), axis=1, keepdims=True)
    h = (x - mu) * lax.rsqrt(var + 1e-5) * g_ref[...] + b_ref[...]
    dn = (((1,), (1,)), ((), ()))
    q = lax.dot_general(h, wq_ref[...], dn, preferred_element_type=jnp.float32)
    k = lax.dot_general(h, wk_ref[...], dn, preferred_element_type=jnp.float32)
    v = lax.dot_general(h, wv_ref[...], dn, preferred_element_type=jnp.float32)
    q_ref[...] = (q + bq_ref[...]) * (1.0 / math.sqrt(HD))
    k_ref[...] = k + bk_ref[...]
    v_ref[...] = (v + bv_ref[...]).astype(jnp.bfloat16)


def _qkv(x, ln1_g, ln1_b, Wq, bq, Wk, bk, Wv, bv):
    grid = (N // _ROWBLK,)
    full = pl.BlockSpec((C, C), lambda i: (0, 0))
    vec = pl.BlockSpec((1, C), lambda i: (0, 0))
    blk = pl.BlockSpec((_ROWBLK, C), lambda i: (i, 0))
    return pl.pallas_call(
        _qkv_body,
        grid=grid,
        in_specs=[blk, vec, vec, full, vec, full, vec, full, vec],
        out_specs=[blk, blk, blk],
        out_shape=[jax.ShapeDtypeStruct((N, C), jnp.float32),
                   jax.ShapeDtypeStruct((N, C), jnp.float32),
                   jax.ShapeDtypeStruct((N, C), jnp.bfloat16)],
    )(x, ln1_g, ln1_b, Wq, bq, Wk, bk, Wv, bv)


@functools.partial(
    pl.kernel,
    out_type=(
        jax.ShapeDtypeStruct((H * _EP,), jnp.float32),
        jax.ShapeDtypeStruct((_NW, H * 16), jnp.float32),
    ),
    mesh=_mesh,
    scratch_types=[
        [pltpu.VMEM((_RC,), jnp.int32) for _ in range(2)],
        [pltpu.VMEM((_EB, CP), jnp.float32) for _ in range(4)],
        [pltpu.VMEM((_WCH,), jnp.float32) for _ in range(2)],
        pltpu.VMEM((H * 16,), jnp.float32),
        [pltpu.SemaphoreType.DMA for _ in range(2)],
    ],
    compiler_params=_scp,
)
def _edge_w(rc_hbm, q_hbm, k_hbm, w_hbm, dpart_hbm,
            rcs, qks, wbufs, dacc, gsems):
    cid = lax.axis_index("c")
    sid = lax.axis_index("s")
    wid = cid * _NS + sid
    zero16 = jnp.zeros((16,), jnp.float32)
    for h in range(H):
        dacc[pl.ds(h * 16, 16)] = zero16
    lane = lax.iota(jnp.int32, 16)
    crot = [(lane + t) & 15 for t in range(16)]
    nch = jnp.where(cid == 0, _NCA, _NCB)
    cg0 = jnp.where(cid == 0, sid * _NCA, _NS * _NCA + sid * _NCB)

    def issue(c, vs):
        pltpu.sync_copy(rc_hbm.at[pl.ds((cg0 + c) * _RC, _RC)], rcs[vs])
        pltpu.async_copy(q_hbm.at[rcs[vs].at[pl.ds(0, _EB)]],
                         qks[2 * vs], gsems[vs])
        pltpu.async_copy(k_hbm.at[rcs[vs].at[pl.ds(_EB, _EB)]],
                         qks[2 * vs + 1], gsems[vs])

    def gath_wait(vs):
        pltpu.make_async_copy(q_hbm.at[rcs[vs].at[pl.ds(0, _EB)]],
                              qks[2 * vs], gsems[vs]).wait()
        pltpu.make_async_copy(k_hbm.at[rcs[vs].at[pl.ds(_EB, _EB)]],
                              qks[2 * vs + 1], gsems[vs]).wait()

    def compute(c, vs):
        qrows = qks[2 * vs]
        krows = qks[2 * vs + 1]
        wbuf = wbufs[vs]
        ebase = (cg0 + c) * _EB

        def grp(g, c2):
            eloc = g * 16 + lane
            mask = (ebase + eloc) < E
            for h in range(H):
                a0 = zero16
                a1 = zero16
                a2 = zero16
                a3 = zero16
                for d in range(CP // H):
                    cvec = crot[d] + (h * (CP // H))
                    qg = plsc.load_gather(qrows, [eloc, cvec])
                    kg = plsc.load_gather(krows, [eloc, cvec])
                    prod = (plsc.bitcast(qg, jnp.bfloat16)
                            * plsc.bitcast(kg, jnp.bfloat16))
                    pa, pb = plsc.unpack(prod,
                                         format=plsc.PackFormat.INTERLEAVED)
                    if d % 2 == 0:
                        a0 = a0 + pa
                        a1 = a1 + pb
                    else:
                        a2 = a2 + pa
                        a3 = a3 + pb
                acc = (a0 + a1) + (a2 + a3)
                wv = jnp.where(mask, jnp.exp(acc), 0.0)
                sl = pl.ds(h * 16, 16)
                dacc[sl] = dacc[sl] + wv
                wbuf[pl.ds(h * _EB + g * 16, 16)] = wv
            return c2

        lax.fori_loop(0, _EB // 16, grp, 0)
        pltpu.sync_copy(wbuf, w_hbm.at[pl.ds((cg0 + c) * _WCH, _WCH)])

    issue(0, 0)
    issue(1, 1)

    def body(j, carry):
        gath_wait(0)
        compute(2 * j, 0)
        issue((2 * j + 2) % nch, 0)
        gath_wait(1)
        compute(2 * j + 1, 1)
        issue((2 * j + 3) % nch, 1)
        return carry

    lax.fori_loop(0, nch // 2, body, 0)
    gath_wait(0)
    gath_wait(1)
    pltpu.sync_copy(dacc, dpart_hbm.at[wid])


@functools.partial(
    pl.kernel,
    out_type=jax.ShapeDtypeStruct((_NC * _RSC, C), jnp.bfloat16),
    mesh=_mesh,
    scratch_types=[
        [pltpu.VMEM((_RC,), jnp.int32) for _ in range(2)],
        [pltpu.VMEM((_WCH,), jnp.float32) for _ in range(2)],
        [pltpu.VMEM((_EB,), jnp.int32) for _ in range(2)],
        [pltpu.VMEM((_EB, C), jnp.bfloat16) for _ in range(2)],
        [pltpu.VMEM((_EB, C), jnp.bfloat16) for _ in range(2)],
        pltpu.VMEM((16, C), jnp.bfloat16),
        pltpu.VMEM_SHARED((_RSC, C), jnp.bfloat16),
        [pltpu.SemaphoreType.DMA for _ in range(2)],
        [pltpu.SemaphoreType.DMA for _ in range(2)],
    ],
    compiler_params=_scp,
)
def _edge_scatter(rc_hbm, v_hbm, w_hbm, out_hbm,
                  rcs, wchs, rowvs, vrows, wvs, zbuf, acc,
                  gsems, ssems):
    cid = lax.axis_index("c")
    sid = lax.axis_index("s")
    lane = lax.iota(jnp.int32, 16)
    zero32 = jnp.zeros((32,), jnp.bfloat16)
    nch = jnp.where(cid == 0, _NCA, _NCB)
    cg0 = jnp.where(cid == 0, sid * _NCA, _NS * _NCA + sid * _NCB)

    def issue(c, vs):
        pltpu.sync_copy(rc_hbm.at[pl.ds((cg0 + c) * _RC, _RC)], rcs[vs])
        pltpu.sync_copy(w_hbm.at[pl.ds((cg0 + c) * _WCH, _WCH)], wchs[vs])
        pltpu.async_copy(v_hbm.at[rcs[vs].at[pl.ds(_EB, _EB)]],
                         vrows[vs], gsems[vs])

    def gath_wait(vs):
        pltpu.make_async_copy(v_hbm.at[rcs[vs].at[pl.ds(_EB, _EB)]],
                              vrows[vs], gsems[vs]).wait()

    def scat_wait(vs):
        pltpu.make_async_copy(wvs[vs], acc.at[rowvs[vs]], ssems[vs]).wait()

    def process(vs):
        for t in range(_EB // 16):
            rowvs[vs][pl.ds(t * 16, 16)] = rcs[vs][pl.ds(t * 16, 16)]
        wch = wchs[vs]
        wvbuf = wvs[vs]
        vr = vrows[vs]

        def grp(g, c2):
            for h in range(H):
                w16 = wch[pl.ds(h * _EB + g * 16, 16)]
                for e16 in range(16):
                    e = g * 16 + e16
                    wb = jnp.broadcast_to(w16[e16], (16,))
                    wbb = plsc.pack(wb, wb, format=plsc.PackFormat.INTERLEAVED)
                    sl = pl.ds(h * HD, 32)
                    wvbuf[e, sl] = vr[e, sl] * wbb
            return c2

        lax.fori_loop(0, _EB // 16, grp, 0)
        pltpu.async_copy(wvbuf, acc.at[rowvs[vs]], ssems[vs], add=True)

    for r in range(16):
        for j2 in range(C // 32):
            zbuf[r, pl.ds(j2 * 32, 32)] = zero32

    issue(0, 0)
    issue(1, 1)

    def zinit(t, carry):
        pltpu.sync_copy(zbuf, acc.at[pl.ds(sid * _RPT + t * 16, 16)])
        return carry

    lax.fori_loop(0, _RPT // 16, zinit, 0)
    plsc.subcore_barrier()

    def body(j, carry):
        @pl.when(j > 0)
        def _():
            scat_wait(0)
            scat_wait(1)

        gath_wait(0)
        process(0)
        issue((2 * j + 2) % nch, 0)
        gath_wait(1)
        process(1)
        issue((2 * j + 3) % nch, 1)
        return carry

    lax.fori_loop(0, nch // 2, body, 0)
    gath_wait(0)
    gath_wait(1)
    scat_wait(0)
    scat_wait(1)
    plsc.subcore_barrier()

    def wout(t, carry):
        sl = pl.ds(sid * _RPT + t * 16, 16)
        pltpu.sync_copy(acc.at[sl],
                        out_hbm.at[pl.ds(cid * _RSC + sid * _RPT + t * 16, 16)])
        return carry

    lax.fori_loop(0, _RPT // 16, wout, 0)


def _erf(x):
    a1, a2, a3 = 0.254829592, -0.284496736, 1.421413741
    a4, a5, p = -1.453152027, 1.061405429, 0.3275911
    s = jnp.sign(x)
    z = jnp.abs(x)
    t = 1.0 / (1.0 + p * z)
    y = 1.0 - (((((a5 * t + a4) * t) + a3) * t + a2) * t + a1) * t * jnp.exp(-z * z)
    return s * y


def _gelu(x):
    return 0.5 * x * (1.0 + _erf(x * (1.0 / math.sqrt(2.0))))


def _epi_body(a0_ref, a1_ref, x_ref, dp_ref, grp_ref, sel_ref, wo_ref, bo_ref,
              g2_ref, b2_ref, w1_ref, b1_ref, w2_ref, bias2_ref, out_ref):
    dn = (((1,), (1,)), ((), ()))
    dnr = (((1,), (0,)), ((), ()))
    dsum = jnp.sum(dp_ref[...], axis=0, keepdims=True)
    den8 = lax.dot_general(dsum, grp_ref[...], dnr,
                           preferred_element_type=jnp.float32)
    svec = lax.dot_general(1.0 / den8, sel_ref[...], dnr,
                           preferred_element_type=jnp.float32)
    att = (a0_ref[...].astype(jnp.float32)
           + a1_ref[...].astype(jnp.float32)) * svec
    o = lax.dot_general(att, wo_ref[...], dn,
                        preferred_element_type=jnp.float32) + bo_ref[...]
    o = o + x_ref[...]
    mu = jnp.mean(o, axis=1, keepdims=True)
    var = jnp.mean((o - mu) * (o - mu), axis=1, keepdims=True)
    t = (o - mu) * lax.rsqrt(var + 1e-5) * g2_ref[...] + b2_ref[...]
    u = lax.dot_general(t, w1_ref[...], dn,
                        preferred_element_type=jnp.float32) + b1_ref[...]
    u = _gelu(u)
    y = lax.dot_general(u, w2_ref[...], dn,
                        preferred_element_type=jnp.float32) + bias2_ref[...]
    out_ref[...] = y + o


def _epilogue(a0, a1, x, dparts, grp, sel, Wo, bo, ln2_g, ln2_b, W1, b1, W2, b2):
    grid = (N // _ROWBLK,)
    blk = pl.BlockSpec((_ROWBLK, C), lambda i: (i, 0))
    vec = pl.BlockSpec((1, C), lambda i: (0, 0))
    return pl.pallas_call(
        _epi_body,
        grid=grid,
        in_specs=[
            blk, blk, blk,
            pl.BlockSpec((_NW, H * 16), lambda i: (0, 0)),
            pl.BlockSpec((H * 16, H), lambda i: (0, 0)),
            pl.BlockSpec((H, C), lambda i: (0, 0)),
            pl.BlockSpec((C, C), lambda i: (0, 0)),
            vec, vec, vec,
            pl.BlockSpec((4 * C, C), lambda i: (0, 0)),
            pl.BlockSpec((1, 4 * C), lambda i: (0, 0)),
            pl.BlockSpec((C, 4 * C), lambda i: (0, 0)),
            vec,
        ],
        out_specs=blk,
        out_shape=jax.ShapeDtypeStruct((N, C), jnp.float32),
    )(a0, a1, x, dparts, grp, sel, Wo, bo, ln2_g, ln2_b, W1, b1, W2, b2)


def _selectors():
    sel = jnp.repeat(jnp.eye(H, dtype=jnp.float32), HD, axis=1)
    grp = jnp.repeat(jnp.eye(H, dtype=jnp.float32), 16, axis=1).T
    return grp, sel


def kernel(x, edge_index, Wq, bq, Wk, bk, Wv, bv, Wo, bo,
           ln1_g, ln1_b, ln2_g, ln2_b, W1, b1, W2, b2):
    pad = jnp.zeros((_EP - E,), jnp.int32)
    row = jnp.concatenate([edge_index[0], pad])
    col = jnp.concatenate([edge_index[1], pad])
    rc = jnp.concatenate([row.reshape(-1, _EB), col.reshape(-1, _EB)],
                         axis=1).reshape(-1)
    q, k, v = _qkv(x, ln1_g.reshape(1, C), ln1_b.reshape(1, C),
                   Wq, bq.reshape(1, C), Wk, bk.reshape(1, C),
                   Wv, bv.reshape(1, C))
    qp = lax.bitcast_convert_type(
        q.astype(jnp.bfloat16).reshape(N, CP, 2), jnp.float32)
    kp = lax.bitcast_convert_type(
        k.astype(jnp.bfloat16).reshape(N, CP, 2), jnp.float32)
    w, dparts = _edge_w(rc, qp, kp)
    out_pad = _edge_scatter(rc, v, w)
    a0 = out_pad[:N]
    a1 = out_pad[_RSC:_RSC + N]
    grp, sel = _selectors()
    return _epilogue(a0, a1, x, dparts, grp, sel,
                     Wo, bo.reshape(1, C), ln2_g.reshape(1, C),
                     ln2_b.reshape(1, C), W1, b1.reshape(1, 4 * C),
                     W2, b2.reshape(1, C))


# speedup vs baseline: 1.1270x; 1.0053x over previous
import functools
import math

import jax
import jax.numpy as jnp
from jax import lax
from jax.experimental import pallas as pl
from jax.experimental.pallas import tpu as pltpu
from jax.experimental.pallas import tpu_sc as plsc

N = 10000
E = 160000
C = 256
H = 8
HD = 32
CP = C // 2

_NC = 2
_NS = 16
_NW = _NC * _NS

_EB = 80
_EWP = 5120
_EP = _NW * _EWP
_NCH = _EWP // _EB
_NCA = 88
_NCB = 2 * _NCH - _NCA
_WCH = _EB * H
_RC = 2 * _EB
_RSC = 10240
_RPT = _RSC // _NS

_ROWBLK = 2000

_mesh = plsc.VectorSubcoreMesh(core_axis_name="c", subcore_axis_name="s")
_scp = pltpu.CompilerParams(use_tc_tiling_on_sc=False,
                            needs_layout_passes=False)


def _qkv_body(x_ref, g_ref, b_ref, wq_ref, bq_ref, wk_ref, bk_ref,
              wv_ref, bv_ref, q_ref, k_ref, v_ref):
    x = x_ref[...]
    mu = jnp.mean(x, axis=1, keepdims=True)
    var = jnp.mean((x - mu) * (x - mu), axis=1, keepdims=True)
    h = (x - mu) * lax.rsqrt(var + 1e-5) * g_ref[...] + b_ref[...]
    dn = (((1,), (1,)), ((), ()))
    q = lax.dot_general(h, wq_ref[...], dn, preferred_element_type=jnp.float32)
    k = lax.dot_general(h, wk_ref[...], dn, preferred_element_type=jnp.float32)
    v = lax.dot_general(h, wv_ref[...], dn, preferred_element_type=jnp.float32)
    q_ref[...] = (q + bq_ref[...]) * (1.0 / math.sqrt(HD))
    k_ref[...] = k + bk_ref[...]
    v_ref[...] = (v + bv_ref[...]).astype(jnp.bfloat16)


def _qkv(x, ln1_g, ln1_b, Wq, bq, Wk, bk, Wv, bv):
    grid = (N // _ROWBLK,)
    full = pl.BlockSpec((C, C), lambda i: (0, 0))
    vec = pl.BlockSpec((1, C), lambda i: (0, 0))
    blk = pl.BlockSpec((_ROWBLK, C), lambda i: (i, 0))
    return pl.pallas_call(
        _qkv_body,
        grid=grid,
        in_specs=[blk, vec, vec, full, vec, full, vec, full, vec],
        out_specs=[blk, blk, blk],
        out_shape=[jax.ShapeDtypeStruct((N, C), jnp.float32),
                   jax.ShapeDtypeStruct((N, C), jnp.float32),
                   jax.ShapeDtypeStruct((N, C), jnp.bfloat16)],
    )(x, ln1_g, ln1_b, Wq, bq, Wk, bk, Wv, bv)


@functools.partial(
    pl.kernel,
    out_type=(
        jax.ShapeDtypeStruct((H * _EP,), jnp.float32),
        jax.ShapeDtypeStruct((_NW, H * 16), jnp.float32),
    ),
    mesh=_mesh,
    scratch_types=[
        [pltpu.VMEM((_RC,), jnp.int32) for _ in range(2)],
        [pltpu.VMEM((_EB, CP), jnp.float32) for _ in range(4)],
        [pltpu.VMEM((_WCH,), jnp.float32) for _ in range(2)],
        pltpu.VMEM((H * 16,), jnp.float32),
        [pltpu.SemaphoreType.DMA for _ in range(2)],
    ],
    compiler_params=_scp,
)
def _edge_w(rc_hbm, q_hbm, k_hbm, w_hbm, dpart_hbm,
            rcs, qks, wbufs, dacc, gsems):
    cid = lax.axis_index("c")
    sid = lax.axis_index("s")
    wid = cid * _NS + sid
    zero16 = jnp.zeros((16,), jnp.float32)
    for h in range(H):
        dacc[pl.ds(h * 16, 16)] = zero16
    lane = lax.iota(jnp.int32, 16)
    crot = [(lane + t) & 15 for t in range(16)]
    nch = jnp.where(cid == 0, _NCA, _NCB)
    cg0 = jnp.where(cid == 0, sid * _NCA, _NS * _NCA + sid * _NCB)

    def issue(c, vs):
        pltpu.sync_copy(rc_hbm.at[pl.ds((cg0 + c) * _RC, _RC)], rcs[vs])
        pltpu.async_copy(q_hbm.at[rcs[vs].at[pl.ds(0, _EB)]],
                         qks[2 * vs], gsems[vs])
        pltpu.async_copy(k_hbm.at[rcs[vs].at[pl.ds(_EB, _EB)]],
                         qks[2 * vs + 1], gsems[vs])

    def gath_wait(vs):
        pltpu.make_async_copy(q_hbm.at[rcs[vs].at[pl.ds(0, _EB)]],
                              qks[2 * vs], gsems[vs]).wait()
        pltpu.make_async_copy(k_hbm.at[rcs[vs].at[pl.ds(_EB, _EB)]],
                              qks[2 * vs + 1], gsems[vs]).wait()

    def compute(c, vs):
        qrows = qks[2 * vs]
        krows = qks[2 * vs + 1]
        wbuf = wbufs[vs]
        ebase = (cg0 + c) * _EB

        def grp(g, c2):
            eloc = g * 16 + lane
            mask = (ebase + eloc) < E
            for h in range(H):
                a0 = zero16
                a1 = zero16
                a2 = zero16
                a3 = zero16
                for d in range(CP // H):
                    cvec = crot[d] + (h * (CP // H))
                    qg = plsc.load_gather(qrows, [eloc, cvec])
                    kg = plsc.load_gather(krows, [eloc, cvec])
                    prod = (plsc.bitcast(qg, jnp.bfloat16)
                            * plsc.bitcast(kg, jnp.bfloat16))
                    pa, pb = plsc.unpack(prod,
                                         format=plsc.PackFormat.INTERLEAVED)
                    if d % 2 == 0:
                        a0 = a0 + pa
                        a1 = a1 + pb
                    else:
                        a2 = a2 + pa
                        a3 = a3 + pb
                acc = (a0 + a1) + (a2 + a3)
                wv = jnp.where(mask, jnp.exp(acc), 0.0)
                sl = pl.ds(h * 16, 16)
                dacc[sl] = dacc[sl] + wv
                wbuf[pl.ds(h * _EB + g * 16, 16)] = wv
            return c2

        lax.fori_loop(0, _EB // 16, grp, 0)
        pltpu.sync_copy(wbuf, w_hbm.at[pl.ds((cg0 + c) * _WCH, _WCH)])

    issue(0, 0)
    issue(1, 1)

    def body(j, carry):
        gath_wait(0)
        compute(2 * j, 0)
        issue((2 * j + 2) % nch, 0)
        gath_wait(1)
        compute(2 * j + 1, 1)
        issue((2 * j + 3) % nch, 1)
        return carry

    lax.fori_loop(0, nch // 2, body, 0)
    gath_wait(0)
    gath_wait(1)
    pltpu.sync_copy(dacc, dpart_hbm.at[wid])


@functools.partial(
    pl.kernel,
    out_type=jax.ShapeDtypeStruct((_NC * _RSC, C), jnp.bfloat16),
    mesh=_mesh,
    scratch_types=[
        [pltpu.VMEM((_RC,), jnp.int32) for _ in range(2)],
        [pltpu.VMEM((_WCH,), jnp.float32) for _ in range(2)],
        [pltpu.VMEM((_EB,), jnp.int32) for _ in range(2)],
        [pltpu.VMEM((_EB, C), jnp.bfloat16) for _ in range(2)],
        [pltpu.VMEM((_EB, C), jnp.bfloat16) for _ in range(2)],
        pltpu.VMEM((16, C), jnp.bfloat16),
        pltpu.VMEM_SHARED((_RSC, C), jnp.bfloat16),
        [pltpu.SemaphoreType.DMA for _ in range(2)],
        [pltpu.SemaphoreType.DMA for _ in range(2)],
    ],
    compiler_params=_scp,
)
def _edge_scatter(rc_hbm, v_hbm, w_hbm, out_hbm,
                  rcs, wchs, rowvs, vrows, wvs, zbuf, acc,
                  gsems, ssems):
    cid = lax.axis_index("c")
    sid = lax.axis_index("s")
    lane = lax.iota(jnp.int32, 16)
    zero32 = jnp.zeros((32,), jnp.bfloat16)
    nch = jnp.where(cid == 0, _NCA, _NCB)
    cg0 = jnp.where(cid == 0, sid * _NCA, _NS * _NCA + sid * _NCB)

    def issue(c, vs):
        pltpu.sync_copy(rc_hbm.at[pl.ds((cg0 + c) * _RC, _RC)], rcs[vs])
        pltpu.sync_copy(w_hbm.at[pl.ds((cg0 + c) * _WCH, _WCH)], wchs[vs])
        pltpu.async_copy(v_hbm.at[rcs[vs].at[pl.ds(_EB, _EB)]],
                         vrows[vs], gsems[vs])

    def gath_wait(vs):
        pltpu.make_async_copy(v_hbm.at[rcs[vs].at[pl.ds(_EB, _EB)]],
                              vrows[vs], gsems[vs]).wait()

    def scat_wait(vs):
        pltpu.make_async_copy(wvs[vs], acc.at[rowvs[vs]], ssems[vs]).wait()

    def process(vs):
        for t in range(_EB // 16):
            rowvs[vs][pl.ds(t * 16, 16)] = rcs[vs][pl.ds(t * 16, 16)]
        wch = wchs[vs]
        wvbuf = wvs[vs]
        vr = vrows[vs]

        def grp(g, c2):
            for h in range(H):
                w16 = wch[pl.ds(h * _EB + g * 16, 16)]
                for e16 in range(16):
                    e = g * 16 + e16
                    wb = jnp.broadcast_to(w16[e16], (16,))
                    wbb = plsc.pack(wb, wb, format=plsc.PackFormat.INTERLEAVED)
                    sl = pl.ds(h * HD, 32)
                    wvbuf[e, sl] = vr[e, sl] * wbb
            return c2

        lax.fori_loop(0, _EB // 16, grp, 0)
        pltpu.async_copy(wvbuf, acc.at[rowvs[vs]], ssems[vs], add=True)

    for r in range(16):
        for j2 in range(C // 32):
            zbuf[r, pl.ds(j2 * 32, 32)] = zero32

    issue(0, 0)
    issue(1, 1)

    def zinit(t, carry):
        pltpu.sync_copy(zbuf, acc.at[pl.ds(sid * _RPT + t * 16, 16)])
        return carry

    lax.fori_loop(0, _RPT // 16, zinit, 0)
    plsc.subcore_barrier()

    def body(j, carry):
        @pl.when(j > 0)
        def _():
            scat_wait(0)
            scat_wait(1)

        gath_wait(0)
        process(0)
        issue((2 * j + 2) % nch, 0)
        gath_wait(1)
        process(1)
        issue((2 * j + 3) % nch, 1)
        return carry

    lax.fori_loop(0, nch // 2, body, 0)
    gath_wait(0)
    gath_wait(1)
    scat_wait(0)
    scat_wait(1)
    plsc.subcore_barrier()

    def wout(t, carry):
        sl = pl.ds(sid * _RPT + t * 16, 16)
        pltpu.sync_copy(acc.at[sl],
                        out_hbm.at[pl.ds(cid * _RSC + sid * _RPT + t * 16, 16)])
        return carry

    lax.fori_loop(0, _RPT // 16, wout, 0)


def _erf(x):
    a1, a2, a3 = 0.254829592, -0.284496736, 1.421413741
    a4, a5, p = -1.453152027, 1.061405429, 0.3275911
    s = jnp.sign(x)
    z = jnp.abs(x)
    t = 1.0 / (1.0 + p * z)
    y = 1.0 - (((((a5 * t + a4) * t) + a3) * t + a2) * t + a1) * t * jnp.exp(-z * z)
    return s * y


def _gelu(x):
    return 0.5 * x * (1.0 + _erf(x * (1.0 / math.sqrt(2.0))))


def _epi_body(a0_ref, a1_ref, x_ref, dp_ref, grp_ref, sel_ref, wo_ref, bo_ref,
              g2_ref, b2_ref, w1_ref, b1_ref, w2_ref, bias2_ref, out_ref):
    dn = (((1,), (1,)), ((), ()))
    dnr = (((1,), (0,)), ((), ()))
    dsum = jnp.sum(dp_ref[...], axis=0, keepdims=True)
    den8 = lax.dot_general(dsum, grp_ref[...], dnr,
                           preferred_element_type=jnp.float32)
    svec = lax.dot_general(1.0 / den8, sel_ref[...], dnr,
                           preferred_element_type=jnp.float32)
    att = (a0_ref[...].astype(jnp.float32)
           + a1_ref[...].astype(jnp.float32)) * svec
    o = lax.dot_general(att, wo_ref[...], dn,
                        preferred_element_type=jnp.float32) + bo_ref[...]
    o = o + x_ref[...]
    mu = jnp.mean(o, axis=1, keepdims=True)
    var = jnp.mean((o - mu) * (o - mu), axis=1, keepdims=True)
    t = (o - mu) * lax.rsqrt(var + 1e-5) * g2_ref[...] + b2_ref[...]
    u = lax.dot_general(t, w1_ref[...], dn,
                        preferred_element_type=jnp.float32) + b1_ref[...]
    u = _gelu(u)
    y = lax.dot_general(u, w2_ref[...], dn,
                        preferred_element_type=jnp.float32) + bias2_ref[...]
    out_ref[...] = y + o


def _epilogue(a0, a1, x, dparts, grp, sel, Wo, bo, ln2_g, ln2_b, W1, b1, W2, b2):
    grid = (N // _ROWBLK,)
    blk = pl.BlockSpec((_ROWBLK, C), lambda i: (i, 0))
    vec = pl.BlockSpec((1, C), lambda i: (0, 0))
    return pl.pallas_call(
        _epi_body,
        grid=grid,
        in_specs=[
            blk, blk, blk,
            pl.BlockSpec((_NW, H * 16), lambda i: (0, 0)),
            pl.BlockSpec((H * 16, H), lambda i: (0, 0)),
            pl.BlockSpec((H, C), lambda i: (0, 0)),
            pl.BlockSpec((C, C), lambda i: (0, 0)),
            vec, vec, vec,
            pl.BlockSpec((4 * C, C), lambda i: (0, 0)),
            pl.BlockSpec((1, 4 * C), lambda i: (0, 0)),
            pl.BlockSpec((C, 4 * C), lambda i: (0, 0)),
            vec,
        ],
        out_specs=blk,
        out_shape=jax.ShapeDtypeStruct((N, C), jnp.float32),
    )(a0, a1, x, dparts, grp, sel, Wo, bo, ln2_g, ln2_b, W1, b1, W2, b2)


def _selectors():
    sel = jnp.repeat(jnp.eye(H, dtype=jnp.float32), HD, axis=1)
    grp = jnp.repeat(jnp.eye(H, dtype=jnp.float32), 16, axis=1).T
    return grp, sel


def kernel(x, edge_index, Wq, bq, Wk, bk, Wv, bv, Wo, bo,
           ln1_g, ln1_b, ln2_g, ln2_b, W1, b1, W2, b2):
    pad = jnp.zeros((_EP - E,), jnp.int32)
    row = jnp.concatenate([edge_index[0], pad])
    col = jnp.concatenate([edge_index[1], pad])
    rc = jnp.concatenate([row.reshape(-1, _EB), col.reshape(-1, _EB)],
                         axis=1).reshape(-1)
    q, k, v = _qkv(x, ln1_g.reshape(1, C), ln1_b.reshape(1, C),
                   Wq, bq.reshape(1, C), Wk, bk.reshape(1, C),
                   Wv, bv.reshape(1, C))
    qp = lax.bitcast_convert_type(
        q.astype(jnp.bfloat16).reshape(N, CP, 2), jnp.float32)
    kp = lax.bitcast_convert_type(
        k.astype(jnp.bfloat16).reshape(N, CP, 2), jnp.float32)
    w, dparts = _edge_w(rc, qp, kp)
    out_pad = _edge_scatter(rc, v, w)
    a0 = out_pad[:N]
    a1 = out_pad[_RSC:_RSC + N]
    grp, sel = _selectors()
    return _epilogue(a0, a1, x, dparts, grp, sel,
                     Wo, bo.reshape(1, C), ln2_g.reshape(1, C),
                     ln2_b.reshape(1, C), W1, b1.reshape(1, 4 * C),
                     W2, b2.reshape(1, C))


# speedup vs baseline: 1.1346x; 1.0067x over previous
import functools
import math

import jax
import jax.numpy as jnp
from jax import lax
from jax.experimental import pallas as pl
from jax.experimental.pallas import tpu as pltpu
from jax.experimental.pallas import tpu_sc as plsc

N = 10000
E = 160000
C = 256
H = 8
HD = 32
CP = C // 2

_NC = 2
_NS = 16
_NW = _NC * _NS

_EB = 80
_EWP = 5120
_EP = _NW * _EWP
_NCH = _EWP // _EB
_NCA = 96
_NCB = 2 * _NCH - _NCA
_WCH = _EB * H
_RC = 2 * _EB
_RSC = 10240
_RPT = _RSC // _NS

_ROWBLK = 2000

_mesh = plsc.VectorSubcoreMesh(core_axis_name="c", subcore_axis_name="s")
_scp = pltpu.CompilerParams(use_tc_tiling_on_sc=False,
                            needs_layout_passes=False)


def _qkv_body(x_ref, g_ref, b_ref, wq_ref, bq_ref, wk_ref, bk_ref,
              wv_ref, bv_ref, q_ref, k_ref, v_ref):
    x = x_ref[...]
    mu = jnp.mean(x, axis=1, keepdims=True)
    var = jnp.mean((x - mu) * (x - mu), axis=1, keepdims=True)
    h = (x - mu) * lax.rsqrt(var + 1e-5) * g_ref[...] + b_ref[...]
    dn = (((1,), (1,)), ((), ()))
    q = lax.dot_general(h, wq_ref[...], dn, preferred_element_type=jnp.float32)
    k = lax.dot_general(h, wk_ref[...], dn, preferred_element_type=jnp.float32)
    v = lax.dot_general(h, wv_ref[...], dn, preferred_element_type=jnp.float32)
    q_ref[...] = (q + bq_ref[...]) * (1.0 / math.sqrt(HD))
    k_ref[...] = k + bk_ref[...]
    v_ref[...] = (v + bv_ref[...]).astype(jnp.bfloat16)


def _qkv(x, ln1_g, ln1_b, Wq, bq, Wk, bk, Wv, bv):
    grid = (N // _ROWBLK,)
    full = pl.BlockSpec((C, C), lambda i: (0, 0))
    vec = pl.BlockSpec((1, C), lambda i: (0, 0))
    blk = pl.BlockSpec((_ROWBLK, C), lambda i: (i, 0))
    return pl.pallas_call(
        _qkv_body,
        grid=grid,
        in_specs=[blk, vec, vec, full, vec, full, vec, full, vec],
        out_specs=[blk, blk, blk],
        out_shape=[jax.ShapeDtypeStruct((N, C), jnp.float32),
                   jax.ShapeDtypeStruct((N, C), jnp.float32),
                   jax.ShapeDtypeStruct((N, C), jnp.bfloat16)],
    )(x, ln1_g, ln1_b, Wq, bq, Wk, bk, Wv, bv)


@functools.partial(
    pl.kernel,
    out_type=(
        jax.ShapeDtypeStruct((H * _EP,), jnp.float32),
        jax.ShapeDtypeStruct((_NW, H * 16), jnp.float32),
    ),
    mesh=_mesh,
    scratch_types=[
        [pltpu.VMEM((_RC,), jnp.int32) for _ in range(2)],
        [pltpu.VMEM((_EB, CP), jnp.float32) for _ in range(4)],
        [pltpu.VMEM((_WCH,), jnp.float32) for _ in range(2)],
        pltpu.VMEM((H * 16,), jnp.float32),
        [pltpu.SemaphoreType.DMA for _ in range(2)],
    ],
    compiler_params=_scp,
)
def _edge_w(rc_hbm, q_hbm, k_hbm, w_hbm, dpart_hbm,
            rcs, qks, wbufs, dacc, gsems):
    cid = lax.axis_index("c")
    sid = lax.axis_index("s")
    wid = cid * _NS + sid
    zero16 = jnp.zeros((16,), jnp.float32)
    for h in range(H):
        dacc[pl.ds(h * 16, 16)] = zero16
    lane = lax.iota(jnp.int32, 16)
    crot = [(lane + t) & 15 for t in range(16)]
    nch = jnp.where(cid == 0, _NCA, _NCB)
    cg0 = jnp.where(cid == 0, sid * _NCA, _NS * _NCA + sid * _NCB)

    def issue(c, vs):
        pltpu.sync_copy(rc_hbm.at[pl.ds((cg0 + c) * _RC, _RC)], rcs[vs])
        pltpu.async_copy(q_hbm.at[rcs[vs].at[pl.ds(0, _EB)]],
                         qks[2 * vs], gsems[vs])
        pltpu.async_copy(k_hbm.at[rcs[vs].at[pl.ds(_EB, _EB)]],
                         qks[2 * vs + 1], gsems[vs])

    def gath_wait(vs):
        pltpu.make_async_copy(q_hbm.at[rcs[vs].at[pl.ds(0, _EB)]],
                              qks[2 * vs], gsems[vs]).wait()
        pltpu.make_async_copy(k_hbm.at[rcs[vs].at[pl.ds(_EB, _EB)]],
                              qks[2 * vs + 1], gsems[vs]).wait()

    def compute(c, vs):
        qrows = qks[2 * vs]
        krows = qks[2 * vs + 1]
        wbuf = wbufs[vs]
        ebase = (cg0 + c) * _EB

        def grp(g, c2):
            eloc = g * 16 + lane
            mask = (ebase + eloc) < E
            for h in range(H):
                a0 = zero16
                a1 = zero16
                a2 = zero16
                a3 = zero16
                for d in range(CP // H):
                    cvec = crot[d] + (h * (CP // H))
                    qg = plsc.load_gather(qrows, [eloc, cvec])
                    kg = plsc.load_gather(krows, [eloc, cvec])
                    prod = (plsc.bitcast(qg, jnp.bfloat16)
                            * plsc.bitcast(kg, jnp.bfloat16))
                    pa, pb = plsc.unpack(prod,
                                         format=plsc.PackFormat.INTERLEAVED)
                    if d % 2 == 0:
                        a0 = a0 + pa
                        a1 = a1 + pb
                    else:
                        a2 = a2 + pa
                        a3 = a3 + pb
                acc = (a0 + a1) + (a2 + a3)
                wv = jnp.where(mask, jnp.exp(acc), 0.0)
                sl = pl.ds(h * 16, 16)
                dacc[sl] = dacc[sl] + wv
                wbuf[pl.ds(h * _EB + g * 16, 16)] = wv
            return c2

        lax.fori_loop(0, _EB // 16, grp, 0)
        pltpu.sync_copy(wbuf, w_hbm.at[pl.ds((cg0 + c) * _WCH, _WCH)])

    issue(0, 0)
    issue(1, 1)

    def body(j, carry):
        gath_wait(0)
        compute(2 * j, 0)
        issue((2 * j + 2) % nch, 0)
        gath_wait(1)
        compute(2 * j + 1, 1)
        issue((2 * j + 3) % nch, 1)
        return carry

    lax.fori_loop(0, nch // 2, body, 0)
    gath_wait(0)
    gath_wait(1)
    pltpu.sync_copy(dacc, dpart_hbm.at[wid])


@functools.partial(
    pl.kernel,
    out_type=jax.ShapeDtypeStruct((_NC * _RSC, C), jnp.bfloat16),
    mesh=_mesh,
    scratch_types=[
        [pltpu.VMEM((_RC,), jnp.int32) for _ in range(2)],
        [pltpu.VMEM((_WCH,), jnp.float32) for _ in range(2)],
        [pltpu.VMEM((_EB,), jnp.int32) for _ in range(2)],
        [pltpu.VMEM((_EB, C), jnp.bfloat16) for _ in range(2)],
        [pltpu.VMEM((_EB, C), jnp.bfloat16) for _ in range(2)],
        pltpu.VMEM((16, C), jnp.bfloat16),
        pltpu.VMEM_SHARED((_RSC, C), jnp.bfloat16),
        [pltpu.SemaphoreType.DMA for _ in range(2)],
        [pltpu.SemaphoreType.DMA for _ in range(2)],
    ],
    compiler_params=_scp,
)
def _edge_scatter(rc_hbm, v_hbm, w_hbm, out_hbm,
                  rcs, wchs, rowvs, vrows, wvs, zbuf, acc,
                  gsems, ssems):
    cid = lax.axis_index("c")
    sid = lax.axis_index("s")
    lane = lax.iota(jnp.int32, 16)
    zero32 = jnp.zeros((32,), jnp.bfloat16)
    nch = jnp.where(cid == 0, _NCA, _NCB)
    cg0 = jnp.where(cid == 0, sid * _NCA, _NS * _NCA + sid * _NCB)

    def issue(c, vs):
        pltpu.sync_copy(rc_hbm.at[pl.ds((cg0 + c) * _RC, _RC)], rcs[vs])
        pltpu.sync_copy(w_hbm.at[pl.ds((cg0 + c) * _WCH, _WCH)], wchs[vs])
        pltpu.async_copy(v_hbm.at[rcs[vs].at[pl.ds(_EB, _EB)]],
                         vrows[vs], gsems[vs])

    def gath_wait(vs):
        pltpu.make_async_copy(v_hbm.at[rcs[vs].at[pl.ds(_EB, _EB)]],
                              vrows[vs], gsems[vs]).wait()

    def scat_wait(vs):
        pltpu.make_async_copy(wvs[vs], acc.at[rowvs[vs]], ssems[vs]).wait()

    def process(vs):
        for t in range(_EB // 16):
            rowvs[vs][pl.ds(t * 16, 16)] = rcs[vs][pl.ds(t * 16, 16)]
        wch = wchs[vs]
        wvbuf = wvs[vs]
        vr = vrows[vs]

        def grp(g, c2):
            for h in range(H):
                w16 = wch[pl.ds(h * _EB + g * 16, 16)]
                for e16 in range(16):
                    e = g * 16 + e16
                    wb = jnp.broadcast_to(w16[e16], (16,))
                    wbb = plsc.pack(wb, wb, format=plsc.PackFormat.INTERLEAVED)
                    sl = pl.ds(h * HD, 32)
                    wvbuf[e, sl] = vr[e, sl] * wbb
            return c2

        lax.fori_loop(0, _EB // 16, grp, 0)
        pltpu.async_copy(wvbuf, acc.at[rowvs[vs]], ssems[vs], add=True)

    for r in range(16):
        for j2 in range(C // 32):
            zbuf[r, pl.ds(j2 * 32, 32)] = zero32

    issue(0, 0)
    issue(1, 1)

    def zinit(t, carry):
        pltpu.sync_copy(zbuf, acc.at[pl.ds(sid * _RPT + t * 16, 16)])
        return carry

    lax.fori_loop(0, _RPT // 16, zinit, 0)
    plsc.subcore_barrier()

    def body(j, carry):
        @pl.when(j > 0)
        def _():
            scat_wait(0)
            scat_wait(1)

        gath_wait(0)
        process(0)
        issue((2 * j + 2) % nch, 0)
        gath_wait(1)
        process(1)
        issue((2 * j + 3) % nch, 1)
        return carry

    lax.fori_loop(0, nch // 2, body, 0)
    gath_wait(0)
    gath_wait(1)
    scat_wait(0)
    scat_wait(1)
    plsc.subcore_barrier()

    def wout(t, carry):
        sl = pl.ds(sid * _RPT + t * 16, 16)
        pltpu.sync_copy(acc.at[sl],
                        out_hbm.at[pl.ds(cid * _RSC + sid * _RPT + t * 16, 16)])
        return carry

    lax.fori_loop(0, _RPT // 16, wout, 0)


def _erf(x):
    a1, a2, a3 = 0.254829592, -0.284496736, 1.421413741
    a4, a5, p = -1.453152027, 1.061405429, 0.3275911
    s = jnp.sign(x)
    z = jnp.abs(x)
    t = 1.0 / (1.0 + p * z)
    y = 1.0 - (((((a5 * t + a4) * t) + a3) * t + a2) * t + a1) * t * jnp.exp(-z * z)
    return s * y


def _gelu(x):
    return 0.5 * x * (1.0 + _erf(x * (1.0 / math.sqrt(2.0))))


def _epi_body(a0_ref, a1_ref, x_ref, dp_ref, grp_ref, sel_ref, wo_ref, bo_ref,
              g2_ref, b2_ref, w1_ref, b1_ref, w2_ref, bias2_ref, out_ref):
    dn = (((1,), (1,)), ((), ()))
    dnr = (((1,), (0,)), ((), ()))
    dsum = jnp.sum(dp_ref[...], axis=0, keepdims=True)
    den8 = lax.dot_general(dsum, grp_ref[...], dnr,
                           preferred_element_type=jnp.float32)
    svec = lax.dot_general(1.0 / den8, sel_ref[...], dnr,
                           preferred_element_type=jnp.float32)
    att = (a0_ref[...].astype(jnp.float32)
           + a1_ref[...].astype(jnp.float32)) * svec
    o = lax.dot_general(att, wo_ref[...], dn,
                        preferred_element_type=jnp.float32) + bo_ref[...]
    o = o + x_ref[...]
    mu = jnp.mean(o, axis=1, keepdims=True)
    var = jnp.mean((o - mu) * (o - mu), axis=1, keepdims=True)
    t = (o - mu) * lax.rsqrt(var + 1e-5) * g2_ref[...] + b2_ref[...]
    u = lax.dot_general(t, w1_ref[...], dn,
                        preferred_element_type=jnp.float32) + b1_ref[...]
    u = _gelu(u)
    y = lax.dot_general(u, w2_ref[...], dn,
                        preferred_element_type=jnp.float32) + bias2_ref[...]
    out_ref[...] = y + o


def _epilogue(a0, a1, x, dparts, grp, sel, Wo, bo, ln2_g, ln2_b, W1, b1, W2, b2):
    grid = (N // _ROWBLK,)
    blk = pl.BlockSpec((_ROWBLK, C), lambda i: (i, 0))
    vec = pl.BlockSpec((1, C), lambda i: (0, 0))
    return pl.pallas_call(
        _epi_body,
        grid=grid,
        in_specs=[
            blk, blk, blk,
            pl.BlockSpec((_NW, H * 16), lambda i: (0, 0)),
            pl.BlockSpec((H * 16, H), lambda i: (0, 0)),
            pl.BlockSpec((H, C), lambda i: (0, 0)),
            pl.BlockSpec((C, C), lambda i: (0, 0)),
            vec, vec, vec,
            pl.BlockSpec((4 * C, C), lambda i: (0, 0)),
            pl.BlockSpec((1, 4 * C), lambda i: (0, 0)),
            pl.BlockSpec((C, 4 * C), lambda i: (0, 0)),
            vec,
        ],
        out_specs=blk,
        out_shape=jax.ShapeDtypeStruct((N, C), jnp.float32),
    )(a0, a1, x, dparts, grp, sel, Wo, bo, ln2_g, ln2_b, W1, b1, W2, b2)


def _selectors():
    sel = jnp.repeat(jnp.eye(H, dtype=jnp.float32), HD, axis=1)
    grp = jnp.repeat(jnp.eye(H, dtype=jnp.float32), 16, axis=1).T
    return grp, sel


def kernel(x, edge_index, Wq, bq, Wk, bk, Wv, bv, Wo, bo,
           ln1_g, ln1_b, ln2_g, ln2_b, W1, b1, W2, b2):
    pad = jnp.zeros((_EP - E,), jnp.int32)
    row = jnp.concatenate([edge_index[0], pad])
    col = jnp.concatenate([edge_index[1], pad])
    rc = jnp.concatenate([row.reshape(-1, _EB), col.reshape(-1, _EB)],
                         axis=1).reshape(-1)
    q, k, v = _qkv(x, ln1_g.reshape(1, C), ln1_b.reshape(1, C),
                   Wq, bq.reshape(1, C), Wk, bk.reshape(1, C),
                   Wv, bv.reshape(1, C))
    qp = lax.bitcast_convert_type(
        q.astype(jnp.bfloat16).reshape(N, CP, 2), jnp.float32)
    kp = lax.bitcast_convert_type(
        k.astype(jnp.bfloat16).reshape(N, CP, 2), jnp.float32)
    w, dparts = _edge_w(rc, qp, kp)
    out_pad = _edge_scatter(rc, v, w)
    a0 = out_pad[:N]
    a1 = out_pad[_RSC:_RSC + N]
    grp, sel = _selectors()
    return _epilogue(a0, a1, x, dparts, grp, sel,
                     Wo, bo.reshape(1, C), ln2_g.reshape(1, C),
                     ln2_b.reshape(1, C), W1, b1.reshape(1, 4 * C),
                     W2, b2.reshape(1, C))
